# Initial kernel scaffold; baseline (speedup 1.0000x reference)
import jax
import jax.numpy as jnp
from jax.experimental import pallas as pl


def kernel(coord, numbers, charge, afv, comb_v_a, comb_v_q, m0_w1, m0_b1, m0_w2, m0_b2, m1_w1, m1_b1, m1_w2, m1_b2, m2_w1, m2_b1, m2_w2, m2_b2):
    raise NotImplementedError("write your pallas kernel here")


# speedup vs baseline: 2.2315x; 2.2315x over previous
import numpy as np
import jax
import jax.numpy as jnp
from jax.experimental import pallas as pl
from jax.experimental.pallas import tpu as pltpu

_N = 64
_S = 16
_K = 8
_C = 32
_RC = 5.0
_SHIFTS = np.linspace(0.8, _RC, _S).astype(np.float32)


def _perm0():
    idx = np.empty(800, np.int32)
    idx[:_C] = np.arange(_C)
    o = _C
    for s in range(_S):
        for c in range(_C):
            idx[o] = _C + c * _S + s
            o += 1
    for k in range(_K):
        for c in range(_C):
            idx[o] = _C + _C * _S + c * _K + k
            o += 1
    return idx


def _perm1():
    idx = np.empty(825, np.int32)
    idx[:800] = _perm0()
    idx[800:] = np.arange(800, 825)
    return idx


def _fwd_kernel(coord_ref, coordt_ref, numsr_ref, numsc_ref, charge_ref,
                afvt_ref, cva_ref, cvq_ref,
                w1t0_ref, b1t0_ref, w2t0_ref, b2t0_ref,
                w1t1_ref, b1t1_ref, w2t1_ref, b2t1_ref,
                w1t2_ref, b1t2_ref, w2t2_ref, b2t2_ref,
                ch_ref, aimt_ref):
    f32 = jnp.float32
    c_col = coord_ref[0]
    c_row = coordt_ref[0]
    numsr = numsr_ref[0]
    numsc = numsc_ref[0]
    Q = charge_ref[0, 0, 0]

    dx = c_col[:, 0:1] - c_row[0:1, :]
    dy = c_col[:, 1:2] - c_row[1:2, :]
    dz = c_col[:, 2:3] - c_row[2:3, :]
    d2 = dx * dx + dy * dy + dz * dz
    d = jnp.sqrt(d2 + 1e-12)
    padr = numsr == 0
    padc = numsc == 0
    ii = jax.lax.broadcasted_iota(jnp.int32, (_N, _N), 0)
    jj = jax.lax.broadcasted_iota(jnp.int32, (_N, _N), 1)
    valid = (~padc) & (~padr) & (ii != jj) & (d < _RC)
    fc = 0.5 * jnp.cos(jnp.pi * jnp.clip(d, 0.0, _RC) / _RC) + 0.5
    fc = jnp.where(valid, fc, 0.0)
    inv = 1.0 / jnp.where(valid, d, 1.0)
    ux = jnp.where(valid, dx * inv, 0.0)
    uy = jnp.where(valid, dy * inv, 0.0)
    uz = jnp.where(valid, dz * inv, 0.0)

    g = [jnp.exp(-4.0 * (d - _SHIFTS[s]) ** 2) * fc for s in range(_S)]
    gka = []
    gkq = []
    for k in range(_K):
        acc_a = g[0] * cva_ref[0, k]
        acc_q = g[0] * cvq_ref[0, k]
        for s in range(1, _S):
            acc_a = acc_a + g[s] * cva_ref[s, k]
            acc_q = acc_q + g[s] * cvq_ref[s, k]
        gka.append(acc_a)
        gkq.append(acc_q)
    u = (ux, uy, uz)
    chans = list(g)
    for dd in range(3):
        for k in range(_K):
            chans.append(u[dd] * gka[k])
    for dd in range(3):
        for k in range(_K):
            chans.append(u[dd] * gkq[k])
    R = jnp.concatenate(chans, axis=1)

    oh = (ii == numsr).astype(f32)
    aT = jnp.dot(afvt_ref[...], oh, preferred_element_type=f32)

    def conv(aT_in, qT_in):
        A = jnp.concatenate([aT_in, qT_in], axis=0)
        out = jnp.dot(A, R, preferred_element_type=f32)
        s_chunks = [out[0:_C, s * _N:(s + 1) * _N] for s in range(_S)]
        v_chunks = []
        for k in range(_K):
            v0 = out[0:_C, (_S + 0 * _K + k) * _N:(_S + 0 * _K + k + 1) * _N]
            v1 = out[0:_C, (_S + 1 * _K + k) * _N:(_S + 1 * _K + k + 1) * _N]
            v2 = out[0:_C, (_S + 2 * _K + k) * _N:(_S + 2 * _K + k + 1) * _N]
            v_chunks.append(v0 * v0 + v1 * v1 + v2 * v2)
        sq = jnp.concatenate(
            [out[_C:_C + 1, s * _N:(s + 1) * _N] for s in range(_S)], axis=0)
        vq_list = []
        for k in range(_K):
            w0 = out[_C:_C + 1, (_S + _K * 3 + 0 * _K + k) * _N:(_S + _K * 3 + 0 * _K + k + 1) * _N]
            w1 = out[_C:_C + 1, (_S + _K * 3 + 1 * _K + k) * _N:(_S + _K * 3 + 1 * _K + k + 1) * _N]
            w2 = out[_C:_C + 1, (_S + _K * 3 + 2 * _K + k) * _N:(_S + _K * 3 + 2 * _K + k + 1) * _N]
            vq_list.append(w0 * w0 + w1 * w1 + w2 * w2)
        vq = jnp.concatenate(vq_list, axis=0)
        return s_chunks, v_chunks, sq, vq

    def mlp(XT, w1t_ref, b1t_ref, w2t_ref, b2t_ref, last_linear):
        h = jax.nn.gelu(jnp.dot(w1t_ref[...], XT, preferred_element_type=f32)
                        + b1t_ref[...])
        o = jnp.dot(w2t_ref[...], h, preferred_element_type=f32) + b2t_ref[...]
        return o if last_linear else jax.nn.gelu(o)

    def zero(x):
        return jnp.where(padr, 0.0, x)

    def nqe(q, f):
        w = f * f
        w = w / (jnp.sum(w) + 1e-6)
        return q + (Q - jnp.sum(q)) * w

    zrow = jnp.zeros((1, _N), f32)
    sc, vc, _, _ = conv(aT, zrow)
    X0 = jnp.concatenate([aT] + sc + vc, axis=0)
    o = zero(mlp(X0, w1t0_ref, b1t0_ref, w2t0_ref, b2t0_ref, True))
    charges = nqe(o[0:1], o[1:2])
    aT = aT + o[2:2 + _C]
    sc, vc, sq, vq = conv(aT, charges)
    X1 = jnp.concatenate([aT] + sc + vc + [charges, sq, vq,
                                           jnp.zeros((7, _N), f32)], axis=0)
    o = zero(mlp(X1, w1t1_ref, b1t1_ref, w2t1_ref, b2t1_ref, False))
    charges = nqe(charges + o[0:1], o[1:2])
    aT = aT + o[2:2 + _C]
    sc, vc, sq, vq = conv(aT, charges)
    X2 = jnp.concatenate([aT] + sc + vc + [charges, sq, vq,
                                           jnp.zeros((7, _N), f32)], axis=0)
    aim = zero(mlp(X2, w1t2_ref, b1t2_ref, w2t2_ref, b2t2_ref, False))
    ch_ref[0] = charges
    aimt_ref[0] = aim


def kernel(coord, numbers, charge, afv, comb_v_a, comb_v_q,
           m0_w1, m0_b1, m0_w2, m0_b2,
           m1_w1, m1_b1, m1_w2, m1_b2,
           m2_w1, m2_b1, m2_w2, m2_b2):
    B, N = coord.shape[0], coord.shape[1]
    f32 = jnp.float32
    coord = coord.astype(f32)
    coord_t = jnp.swapaxes(coord, 1, 2)
    nums = numbers.astype(jnp.int32)
    numsr = nums.reshape(B, 1, N)
    numsc = nums.reshape(B, N, 1)
    chg = charge.astype(f32).reshape(B, 1, 1)
    afvt = afv.astype(f32).T

    p0 = jnp.asarray(_perm0())
    p1 = jnp.asarray(_perm1())
    pad7 = jnp.zeros((7, m1_w1.shape[1]), f32)
    w1t0 = m0_w1[p0].T
    w1t1 = jnp.concatenate([m1_w1[p1], pad7], axis=0).T
    w1t2 = jnp.concatenate([m2_w1[p1], pad7], axis=0).T
    b1t0 = m0_b1.reshape(-1, 1)
    b1t1 = m1_b1.reshape(-1, 1)
    b1t2 = m2_b1.reshape(-1, 1)
    w2t0, w2t1, w2t2 = m0_w2.T, m1_w2.T, m2_w2.T
    b2t0 = m0_b2.reshape(-1, 1)
    b2t1 = m1_b2.reshape(-1, 1)
    b2t2 = m2_b2.reshape(-1, 1)

    def bspec(shape3):
        return pl.BlockSpec(shape3, lambda b: (b, 0, 0))

    def wspec(shape2):
        return pl.BlockSpec(shape2, lambda b: (0, 0))

    in_specs = [
        bspec((1, N, 3)),
        bspec((1, 3, N)),
        bspec((1, 1, N)),
        bspec((1, N, 1)),
        bspec((1, 1, 1)),
        wspec(afvt.shape),
        wspec(comb_v_a.shape),
        wspec(comb_v_q.shape),
        wspec(w1t0.shape), wspec(b1t0.shape), wspec(w2t0.shape), wspec(b2t0.shape),
        wspec(w1t1.shape), wspec(b1t1.shape), wspec(w2t1.shape), wspec(b2t1.shape),
        wspec(w1t2.shape), wspec(b1t2.shape), wspec(w2t2.shape), wspec(b2t2.shape),
    ]
    out_specs = [bspec((1, 1, N)), bspec((1, 256, N))]
    out_shape = [jax.ShapeDtypeStruct((B, 1, N), f32),
                 jax.ShapeDtypeStruct((B, 256, N), f32)]
    ch, aimt = pl.pallas_call(
        _fwd_kernel,
        grid=(B,),
        in_specs=in_specs,
        out_specs=out_specs,
        out_shape=out_shape,
        compiler_params=pltpu.CompilerParams(
            dimension_semantics=("arbitrary",)),
    )(coord, coord_t, numsr, numsc, chg, afvt,
      comb_v_a.astype(f32), comb_v_q.astype(f32),
      w1t0, b1t0, w2t0, b2t0,
      w1t1, b1t1, w2t1, b2t1,
      w1t2, b1t2, w2t2, b2t2)
    return jnp.concatenate([ch.reshape(B, N, 1), jnp.swapaxes(aimt, 1, 2)],
                           axis=-1)
